# baseline (speedup 1.0000x reference)
import functools

import jax
import jax.numpy as jnp
from jax import lax
from jax.experimental import pallas as pl
from jax.experimental.pallas import tpu as pltpu
from jax.experimental.pallas import tpu_sc as plsc

NC = 2
NS = 16
NW = NC * NS
L = 16


@functools.lru_cache(maxsize=None)
def _build(batch: int, embed_dim: int):
    assert embed_dim == 2 * L, "kernel assumes EMBED_DIM == 32"
    assert batch % (8 * NW) == 0
    b_per_w = batch // NW
    n_groups = b_per_w // L

    mesh = plsc.VectorSubcoreMesh(
        core_axis_name="c", subcore_axis_name="s", num_cores=NC, num_subcores=NS
    )

    @functools.partial(
        pl.kernel,
        out_type=jax.ShapeDtypeStruct((batch,), jnp.float32),
        mesh=mesh,
        compiler_params=pltpu.CompilerParams(
            needs_layout_passes=False, use_tc_tiling_on_sc=False
        ),
        scratch_types=[
            pltpu.VMEM((b_per_w,), jnp.int32),
            pltpu.VMEM((b_per_w,), jnp.int32),
            pltpu.VMEM((b_per_w, 2 * L), jnp.float32),
            pltpu.VMEM((b_per_w, 2 * L), jnp.float32),
            pltpu.VMEM((b_per_w,), jnp.float32),
            pltpu.VMEM((b_per_w,), jnp.float32),
            pltpu.VMEM((L * (L + 1),), jnp.float32),
            pltpu.VMEM((b_per_w,), jnp.float32),
            pltpu.VMEM((L,), jnp.float32),
            pltpu.SemaphoreType.DMA,
            pltpu.SemaphoreType.DMA,
            pltpu.SemaphoreType.DMA,
            pltpu.SemaphoreType.DMA,
        ],
    )
    def kern(uid_hbm, iid_hbm, utab_hbm, itab_hbm, ub_hbm, ib_hbm, gb_hbm,
             out_hbm, uidx_v, iidx_v, uemb_v, iemb_v, ubias_v, ibias_v,
             p_v, out_v, gb_v, sem_u, sem_i, sem_ub, sem_ib):
        wid = lax.axis_index("s") * NC + lax.axis_index("c")
        base = wid * b_per_w

        pltpu.sync_copy(uid_hbm.at[pl.ds(base, b_per_w)], uidx_v)
        pltpu.sync_copy(iid_hbm.at[pl.ds(base, b_per_w)], iidx_v)

        cu = pltpu.make_async_copy(utab_hbm.at[uidx_v], uemb_v, sem_u)
        ci = pltpu.make_async_copy(itab_hbm.at[iidx_v], iemb_v, sem_i)
        cub = pltpu.make_async_copy(ub_hbm.at[uidx_v], ubias_v, sem_ub)
        cib = pltpu.make_async_copy(ib_hbm.at[iidx_v], ibias_v, sem_ib)
        cu.start()
        ci.start()
        cub.start()
        cib.start()
        pltpu.sync_copy(gb_hbm, gb_v)
        cu.wait()
        ci.wait()
        cub.wait()
        cib.wait()

        gvec = gb_v[...]
        lanes_scaled = lax.iota(jnp.int32, L) * (L + 1)

        def group(g, _):
            r0 = g * L
            for j in range(L):
                r = r0 + j
                u0 = uemb_v[r, pl.ds(0, L)]
                u1 = uemb_v[r, pl.ds(L, L)]
                i0 = iemb_v[r, pl.ds(0, L)]
                i1 = iemb_v[r, pl.ds(L, L)]
                part = u0 * i0 + u1 * i1
                plsc.store_scatter(p_v, [lanes_scaled + j], part)
            acc = p_v[pl.ds(0, L)]
            for lane in range(1, L):
                acc = acc + p_v[pl.ds(lane * (L + 1), L)]
            score = acc + ubias_v[pl.ds(r0, L)] + ibias_v[pl.ds(r0, L)] + gvec
            out_v[pl.ds(r0, L)] = 1.0 / (1.0 + jnp.exp(-score))
            return ()

        lax.fori_loop(0, n_groups, group, (), unroll=False)
        pltpu.sync_copy(out_v, out_hbm.at[pl.ds(base, b_per_w)])

    return kern


def kernel(inputs, user_table, item_table, user_bias, item_bias, global_bias):
    batch = inputs.shape[0]
    user_ids = inputs[:, 0]
    item_ids = inputs[:, 1]
    ub_flat = user_bias.reshape(-1)
    ib_flat = item_bias.reshape(-1)
    gb_vec = jnp.full((L,), global_bias, dtype=jnp.float32)
    kern = _build(batch, user_table.shape[1])
    return kern(user_ids, item_ids, user_table, item_table, ub_flat, ib_flat,
                gb_vec)


# speedup vs baseline: 1.2918x; 1.2918x over previous
import functools

import jax
import jax.numpy as jnp
from jax import lax
from jax.experimental import pallas as pl
from jax.experimental.pallas import tpu as pltpu
from jax.experimental.pallas import tpu_sc as plsc

NC = 2
NS = 16
NW = NC * NS
L = 16
H = 256


@functools.lru_cache(maxsize=None)
def _build(batch: int, embed_dim: int):
    assert embed_dim == 2 * L, "kernel assumes EMBED_DIM == 32"
    assert batch % (H * NW) == 0
    b_per_w = batch // NW

    mesh = plsc.VectorSubcoreMesh(
        core_axis_name="c", subcore_axis_name="s", num_cores=NC, num_subcores=NS
    )

    @functools.partial(
        pl.kernel,
        out_type=jax.ShapeDtypeStruct((batch,), jnp.float32),
        mesh=mesh,
        compiler_params=pltpu.CompilerParams(needs_layout_passes=False),
        scratch_types=[
            pltpu.VMEM((b_per_w,), jnp.int32),
            pltpu.VMEM((b_per_w,), jnp.int32),
            pltpu.VMEM((H, 2 * L), jnp.float32),
            pltpu.VMEM((H, 2 * L), jnp.float32),
            pltpu.VMEM((b_per_w,), jnp.float32),
            pltpu.VMEM((L * (L + 1),), jnp.float32),
            pltpu.VMEM((b_per_w,), jnp.float32),
            pltpu.SemaphoreType.DMA,
            pltpu.SemaphoreType.DMA,
        ],
    )
    def kern(uid_hbm, iid_hbm, utab_hbm, itab_hbm, bias_hbm, out_hbm,
             uidx_v, iidx_v, uemb_v, iemb_v, bias_v,
             p_v, out_v, sem_u, sem_i):
        wid = lax.axis_index("s") * NC + lax.axis_index("c")
        base = wid * b_per_w

        pltpu.sync_copy(uid_hbm.at[pl.ds(base, b_per_w)], uidx_v)
        pltpu.sync_copy(iid_hbm.at[pl.ds(base, b_per_w)], iidx_v)
        pltpu.sync_copy(bias_hbm.at[pl.ds(base, b_per_w)], bias_v)

        lanes_scaled = lax.iota(jnp.int32, L) * (L + 1)

        def phase(ph, _):
            off = ph * H

            def issue(b, _):
                i0 = b * L
                uvec = uidx_v[pl.ds(off + i0, L)]
                ivec = iidx_v[pl.ds(off + i0, L)]
                for j in range(L):
                    pltpu.make_async_copy(
                        utab_hbm.at[pl.ds(uvec[j], 1)],
                        uemb_v.at[pl.ds(i0 + j, 1)],
                        sem_u,
                    ).start()
                    pltpu.make_async_copy(
                        itab_hbm.at[pl.ds(ivec[j], 1)],
                        iemb_v.at[pl.ds(i0 + j, 1)],
                        sem_i,
                    ).start()
                return ()

            lax.fori_loop(0, H // L, issue, (), unroll=False)
            pltpu.make_async_copy(
                utab_hbm.at[pl.ds(0, H)], uemb_v, sem_u
            ).wait()
            pltpu.make_async_copy(
                itab_hbm.at[pl.ds(0, H)], iemb_v, sem_i
            ).wait()

            def group(g, _):
                r0 = g * L
                for j in range(L):
                    r = r0 + j
                    u0 = uemb_v[r, pl.ds(0, L)]
                    u1 = uemb_v[r, pl.ds(L, L)]
                    i0 = iemb_v[r, pl.ds(0, L)]
                    i1 = iemb_v[r, pl.ds(L, L)]
                    part = u0 * i0 + u1 * i1
                    plsc.store_scatter(p_v, [lanes_scaled + j], part)
                acc = p_v[pl.ds(0, L)]
                for lane in range(1, L):
                    acc = acc + p_v[pl.ds(lane * (L + 1), L)]
                score = acc + bias_v[pl.ds(off + r0, L)]
                out_v[pl.ds(off + r0, L)] = 1.0 / (1.0 + jnp.exp(-score))
                return ()

            lax.fori_loop(0, H // L, group, (), unroll=False)
            return ()

        lax.fori_loop(0, b_per_w // H, phase, (), unroll=False)
        pltpu.sync_copy(out_v, out_hbm.at[pl.ds(base, b_per_w)])

    return kern


def kernel(inputs, user_table, item_table, user_bias, item_bias, global_bias):
    batch = inputs.shape[0]
    user_ids = inputs[:, 0]
    item_ids = inputs[:, 1]
    bias_sum = (
        jnp.take(user_bias[:, 0], user_ids)
        + jnp.take(item_bias[:, 0], item_ids)
        + global_bias
    )
    kern = _build(batch, user_table.shape[1])
    return kern(user_ids, item_ids, user_table, item_table, bias_sum)
